# baseline (speedup 1.0000x reference)
import functools

import jax
import jax.numpy as jnp
from jax import lax
from jax.experimental import pallas as pl
from jax.experimental.pallas import tpu as pltpu
from jax.experimental.pallas import tpu_sc as plsc

BATCH = 16384
EMBED_DIM = 32
NUM_CORES = 2
NUM_SUBCORES = 16
NUM_WORKERS = NUM_CORES * NUM_SUBCORES
ROWS_PER_WORKER = BATCH // NUM_WORKERS
CHUNK = 128
NUM_CHUNKS = ROWS_PER_WORKER // CHUNK
LANES = 16


def _fm_body(uid_hbm, iid_hbm, ut_hbm, it_hbm, out_hbm,
             uidx_v, iidx_v, urows_v, irows_v, out_v, sem):
    wid = lax.axis_index("s") * NUM_CORES + lax.axis_index("c")
    base = wid * NUM_CHUNKS

    pltpu.sync_copy(uid_hbm.at[pl.ds(base, NUM_CHUNKS)], uidx_v)
    pltpu.sync_copy(iid_hbm.at[pl.ds(base, NUM_CHUNKS)], iidx_v)

    copies = []
    for j in range(NUM_CHUNKS):
        dst = urows_v.at[pl.ds(j * CHUNK, CHUNK)]
        copies.append(pltpu.async_copy(ut_hbm.at[uidx_v.at[j]], dst, sem))
        dst = irows_v.at[pl.ds(j * CHUNK, CHUNK)]
        copies.append(pltpu.async_copy(it_hbm.at[iidx_v.at[j]], dst, sem))
    for c in copies:
        c.wait()

    lane = lax.iota(jnp.int32, LANES)

    def group(g, carry):
        acc = jnp.zeros((LANES,), jnp.float32)
        base_r = g * LANES
        for j in range(LANES):
            r = base_r + j
            u = plsc.bitcast(urows_v[r, :], jnp.bfloat16)
            i = plsc.bitcast(irows_v[r, :], jnp.bfloat16)
            ulo, uhi = plsc.unpack(u, format=plsc.PackFormat.INTERLEAVED)
            ilo, ihi = plsc.unpack(i, format=plsc.PackFormat.INTERLEAVED)
            s = (ulo + ilo + ulo * ilo) + (uhi + ihi + uhi * ihi)
            acc = jnp.where(lane == j, jnp.sum(s), acc)
        out_v[pl.ds(g * LANES, LANES)] = acc
        return carry

    lax.fori_loop(0, ROWS_PER_WORKER // LANES, group, 0)
    pltpu.sync_copy(out_v, out_hbm.at[pl.ds(wid * ROWS_PER_WORKER,
                                            ROWS_PER_WORKER)])


def kernel(user_ids, item_ids, user_table, item_table):
    uids = user_ids.astype(jnp.int32).reshape(NUM_WORKERS * NUM_CHUNKS, CHUNK)
    iids = item_ids.astype(jnp.int32).reshape(NUM_WORKERS * NUM_CHUNKS, CHUNK)
    ut16 = jax.lax.bitcast_convert_type(
        user_table.astype(jnp.bfloat16).reshape(NUM_USERS := user_table.shape[0],
                                                EMBED_DIM // 2, 2), jnp.int32)
    it16 = jax.lax.bitcast_convert_type(
        item_table.astype(jnp.bfloat16).reshape(item_table.shape[0],
                                                EMBED_DIM // 2, 2), jnp.int32)
    mesh = plsc.VectorSubcoreMesh(core_axis_name="c", subcore_axis_name="s")
    fm = functools.partial(
        pl.kernel,
        mesh=mesh,
        compiler_params=pltpu.CompilerParams(needs_layout_passes=False,
                                             use_tc_tiling_on_sc=False),
        out_type=jax.ShapeDtypeStruct((BATCH,), jnp.float32),
        scratch_types=[
            pltpu.VMEM((NUM_CHUNKS, CHUNK), jnp.int32),
            pltpu.VMEM((NUM_CHUNKS, CHUNK), jnp.int32),
            pltpu.VMEM((ROWS_PER_WORKER, EMBED_DIM // 2), jnp.int32),
            pltpu.VMEM((ROWS_PER_WORKER, EMBED_DIM // 2), jnp.int32),
            pltpu.VMEM((ROWS_PER_WORKER,), jnp.float32),
            pltpu.SemaphoreType.DMA,
        ],
    )(_fm_body)
    out = fm(uids, iids, ut16, it16)
    return out.reshape(BATCH, 1)


# speedup vs baseline: 7.2618x; 7.2618x over previous
import functools

import jax
import jax.numpy as jnp
from jax import lax
from jax.experimental import pallas as pl
from jax.experimental.pallas import tpu as pltpu
from jax.experimental.pallas import tpu_sc as plsc

BATCH = 16384
EMBED_DIM = 32
NUM_CORES = 2
NUM_SUBCORES = 16
NUM_WORKERS = NUM_CORES * NUM_SUBCORES
ROWS_PER_WORKER = BATCH // NUM_WORKERS
LANES = 16
GROUP = 4
NUM_GROUPS = ROWS_PER_WORKER // GROUP
NUM_PAIRS = NUM_GROUPS // 2
IDS_PAD = ROWS_PER_WORKER + LANES


def _fm_body(uid_hbm, iid_hbm, ut_hbm, it_hbm, out_hbm,
             uids_v, iids_v, slab_v, urows_v, out_v, sem0, sem1):
    wid = lax.axis_index("s") * NUM_CORES + lax.axis_index("c")
    base = wid * ROWS_PER_WORKER

    pltpu.sync_copy(uid_hbm.at[pl.ds(base, ROWS_PER_WORKER)],
                    uids_v.at[pl.ds(0, ROWS_PER_WORKER)])
    pltpu.sync_copy(iid_hbm.at[pl.ds(base, ROWS_PER_WORKER)],
                    iids_v.at[pl.ds(0, ROWS_PER_WORKER)])

    d16a = lax.iota(jnp.int32, LANES)
    d16b = d16a + LANES
    lane = d16a

    def issue(tbl_hbm, ids_v, g, par, sem):
        idv = ids_v[pl.ds(g * GROUP, LANES)]
        for j in range(GROUP):
            r = idv[j]
            cb = pl.multiple_of(r - (r & 127), 128)
            pltpu.async_copy(tbl_hbm.at[:, pl.ds(cb, 128)],
                             slab_v.at[par * GROUP + j], sem)

    def wait_group(par, sem):
        for j in range(GROUP):
            pltpu.make_async_copy(ut_hbm.at[:, pl.ds(0, 128)],
                                  slab_v.at[par * GROUP + j], sem).wait()

    def extract(ids_v, g, par):
        idv = ids_v[pl.ds(g * GROUP, LANES)]
        out = []
        for j in range(GROUP):
            rm = jnp.full((LANES,), idv[j] & 127, jnp.int32)
            zj = jnp.full((LANES,), par * GROUP + j, jnp.int32)
            va = plsc.load_gather(slab_v, [zj, d16a, rm])
            vb = plsc.load_gather(slab_v, [zj, d16b, rm])
            out.append((va, vb))
        return out

    issue(ut_hbm, uids_v, 0, 0, sem0)

    def pass1(t, carry):
        g0 = 2 * t
        g1 = g0 + 1
        issue(ut_hbm, uids_v, g1, 1, sem1)
        wait_group(0, sem0)
        for j, (va, vb) in enumerate(extract(uids_v, g0, 0)):
            r = g0 * GROUP + j
            urows_v[r, pl.ds(0, LANES)] = va
            urows_v[r, pl.ds(LANES, LANES)] = vb

        @pl.when(g0 + 2 < NUM_GROUPS)
        def _():
            issue(ut_hbm, uids_v, g0 + 2, 0, sem0)

        wait_group(1, sem1)
        for j, (va, vb) in enumerate(extract(uids_v, g1, 1)):
            r = g1 * GROUP + j
            urows_v[r, pl.ds(0, LANES)] = va
            urows_v[r, pl.ds(LANES, LANES)] = vb
        return carry

    lax.fori_loop(0, NUM_PAIRS, pass1, 0)

    issue(it_hbm, iids_v, 0, 0, sem0)

    def pass2(t, acc):
        g0 = 2 * t
        g1 = g0 + 1
        half = (t % 2) * (2 * GROUP)
        issue(it_hbm, iids_v, g1, 1, sem1)
        wait_group(0, sem0)
        for j, (ia, ib) in enumerate(extract(iids_v, g0, 0)):
            r = g0 * GROUP + j
            ua = urows_v[r, pl.ds(0, LANES)]
            ub = urows_v[r, pl.ds(LANES, LANES)]
            s = (ua + ia + ua * ia) + (ub + ib + ub * ib)
            acc = jnp.where(lane == half + j, jnp.sum(s), acc)

        @pl.when(g0 + 2 < NUM_GROUPS)
        def _():
            issue(it_hbm, iids_v, g0 + 2, 0, sem0)

        wait_group(1, sem1)
        for j, (ia, ib) in enumerate(extract(iids_v, g1, 1)):
            r = g1 * GROUP + j
            ua = urows_v[r, pl.ds(0, LANES)]
            ub = urows_v[r, pl.ds(LANES, LANES)]
            s = (ua + ia + ua * ia) + (ub + ib + ub * ib)
            acc = jnp.where(lane == half + GROUP + j, jnp.sum(s), acc)

        @pl.when(t % 2 == 1)
        def _():
            out_v[pl.ds((t // 2) * LANES, LANES)] = acc
        return jnp.where(t % 2 == 1, jnp.zeros((LANES,), jnp.float32), acc)

    lax.fori_loop(0, NUM_PAIRS, pass2, jnp.zeros((LANES,), jnp.float32))

    pltpu.sync_copy(out_v, out_hbm.at[pl.ds(base, ROWS_PER_WORKER)])


def kernel(user_ids, item_ids, user_table, item_table):
    uids = user_ids.astype(jnp.int32)
    iids = item_ids.astype(jnp.int32)
    mesh = plsc.VectorSubcoreMesh(core_axis_name="c", subcore_axis_name="s")
    fm = functools.partial(
        pl.kernel,
        mesh=mesh,
        compiler_params=pltpu.CompilerParams(needs_layout_passes=False,
                                             use_tc_tiling_on_sc=True),
        out_type=jax.ShapeDtypeStruct((BATCH,), jnp.float32),
        scratch_types=[
            pltpu.VMEM((IDS_PAD,), jnp.int32),
            pltpu.VMEM((IDS_PAD,), jnp.int32),
            pltpu.VMEM((2 * GROUP, EMBED_DIM, 128), jnp.float32),
            pltpu.VMEM((ROWS_PER_WORKER, EMBED_DIM), jnp.float32),
            pltpu.VMEM((ROWS_PER_WORKER,), jnp.float32),
            pltpu.SemaphoreType.DMA,
            pltpu.SemaphoreType.DMA,
        ],
    )(_fm_body)
    out = fm(uids, iids, user_table.T, item_table.T)
    return out.reshape(BATCH, 1)


# speedup vs baseline: 8.0747x; 1.1119x over previous
import functools

import jax
import jax.numpy as jnp
from jax import lax
from jax.experimental import pallas as pl
from jax.experimental.pallas import tpu as pltpu
from jax.experimental.pallas import tpu_sc as plsc

BATCH = 16384
EMBED_DIM = 32
NUM_CORES = 2
NUM_SUBCORES = 16
NUM_WORKERS = NUM_CORES * NUM_SUBCORES
ROWS_PER_WORKER = BATCH // NUM_WORKERS
LANES = 16
GROUP = 4
NUM_GROUPS = ROWS_PER_WORKER // GROUP
NUM_PAIRS = NUM_GROUPS // 2
IDS_PAD = ROWS_PER_WORKER + LANES


def _fm_body(uid_hbm, iid_hbm, ut_hbm, it_hbm, out_hbm,
             uids_v, iids_v, slab_v, out_v, sem0, sem1):
    wid = lax.axis_index("s") * NUM_CORES + lax.axis_index("c")
    base = wid * ROWS_PER_WORKER

    pltpu.sync_copy(uid_hbm.at[pl.ds(base, ROWS_PER_WORKER)],
                    uids_v.at[pl.ds(0, ROWS_PER_WORKER)])
    pltpu.sync_copy(iid_hbm.at[pl.ds(base, ROWS_PER_WORKER)],
                    iids_v.at[pl.ds(0, ROWS_PER_WORKER)])

    d16a = lax.iota(jnp.int32, LANES)
    d16b = d16a + LANES
    lane = d16a

    def issue(g, par, sem):
        udv = uids_v[pl.ds(g * GROUP, LANES)]
        idv = iids_v[pl.ds(g * GROUP, LANES)]
        for j in range(GROUP):
            ru = udv[j]
            cb = pl.multiple_of(ru - (ru & 127), 128)
            pltpu.async_copy(ut_hbm.at[:, pl.ds(cb, 128)],
                             slab_v.at[par * 2 * GROUP + j], sem)
            ri = idv[j]
            ci = pl.multiple_of(ri - (ri & 127), 128)
            pltpu.async_copy(it_hbm.at[:, pl.ds(ci, 128)],
                             slab_v.at[par * 2 * GROUP + GROUP + j], sem)

    def wait_group(par, sem):
        for j in range(2 * GROUP):
            pltpu.make_async_copy(ut_hbm.at[:, pl.ds(0, 128)],
                                  slab_v.at[par * 2 * GROUP + j], sem).wait()

    def fm(g, par, half, acc):
        udv = uids_v[pl.ds(g * GROUP, LANES)]
        idv = iids_v[pl.ds(g * GROUP, LANES)]
        for j in range(GROUP):
            rum = jnp.full((LANES,), udv[j] & 127, jnp.int32)
            rim = jnp.full((LANES,), idv[j] & 127, jnp.int32)
            zu = jnp.full((LANES,), par * 2 * GROUP + j, jnp.int32)
            zi = jnp.full((LANES,), par * 2 * GROUP + GROUP + j, jnp.int32)
            ua = plsc.load_gather(slab_v, [zu, d16a, rum])
            ub = plsc.load_gather(slab_v, [zu, d16b, rum])
            ia = plsc.load_gather(slab_v, [zi, d16a, rim])
            ib = plsc.load_gather(slab_v, [zi, d16b, rim])
            s = (ua + ia + ua * ia) + (ub + ib + ub * ib)
            acc = jnp.where(lane == half + j, jnp.sum(s), acc)
        return acc

    issue(0, 0, sem0)

    def step(t, acc):
        g0 = 2 * t
        g1 = g0 + 1
        half = (t % 2) * (2 * GROUP)
        issue(g1, 1, sem1)
        wait_group(0, sem0)
        acc = fm(g0, 0, half, acc)

        @pl.when(g0 + 2 < NUM_GROUPS)
        def _():
            issue(g0 + 2, 0, sem0)

        wait_group(1, sem1)
        acc = fm(g1, 1, half + GROUP, acc)

        @pl.when(t % 2 == 1)
        def _():
            out_v[pl.ds((t // 2) * LANES, LANES)] = acc
        return jnp.where(t % 2 == 1, jnp.zeros((LANES,), jnp.float32), acc)

    lax.fori_loop(0, NUM_PAIRS, step, jnp.zeros((LANES,), jnp.float32))

    pltpu.sync_copy(out_v, out_hbm.at[pl.ds(base, ROWS_PER_WORKER)])


def kernel(user_ids, item_ids, user_table, item_table):
    uids = user_ids.astype(jnp.int32)
    iids = item_ids.astype(jnp.int32)
    mesh = plsc.VectorSubcoreMesh(core_axis_name="c", subcore_axis_name="s")
    fm = functools.partial(
        pl.kernel,
        mesh=mesh,
        compiler_params=pltpu.CompilerParams(needs_layout_passes=False,
                                             use_tc_tiling_on_sc=True),
        out_type=jax.ShapeDtypeStruct((BATCH,), jnp.float32),
        scratch_types=[
            pltpu.VMEM((IDS_PAD,), jnp.int32),
            pltpu.VMEM((IDS_PAD,), jnp.int32),
            pltpu.VMEM((4 * GROUP, EMBED_DIM, 128), jnp.float32),
            pltpu.VMEM((ROWS_PER_WORKER,), jnp.float32),
            pltpu.SemaphoreType.DMA,
            pltpu.SemaphoreType.DMA,
        ],
    )(_fm_body)
    out = fm(uids, iids, user_table.T, item_table.T)
    return out.reshape(BATCH, 1)
